# Initial kernel scaffold; baseline (speedup 1.0000x reference)
import jax
import jax.numpy as jnp
from jax import lax
from jax.experimental import pallas as pl
from jax.experimental.pallas import tpu as pltpu
from jax.experimental.pallas import tpu_sc as plsc

B = 4096
L = 200
D = 32
NW = 32
RPW = B // NW
C0 = 104
C1 = L - C0


def _body(text_hbm, len_hbm, idf_hbm, table_hbm, out_hbm,
          text_v, len_v, rows_v, idfs_v, out_v, sem):
    wid = lax.axis_index("s") * 2 + lax.axis_index("c")
    base = wid * RPW

    pltpu.sync_copy(text_hbm.at[pl.ds(base * L, RPW * L)], text_v)
    pltpu.sync_copy(len_hbm.at[pl.ds(base, RPW)], len_v)

    def do_row(r, carry):
        i0 = text_v.at[pl.ds(r * L, C0)]
        i1 = text_v.at[pl.ds(r * L + C0, C1)]
        d0 = pltpu.async_copy(table_hbm.at[i0], rows_v.at[pl.ds(0, C0)], sem)
        d1 = pltpu.async_copy(table_hbm.at[i1], rows_v.at[pl.ds(C0, C1)], sem)
        d2 = pltpu.async_copy(idf_hbm.at[i0], idfs_v.at[pl.ds(0, C0)], sem)
        d3 = pltpu.async_copy(idf_hbm.at[i1], idfs_v.at[pl.ds(C0, C1)], sem)
        d0.wait()
        d1.wait()
        d2.wait()
        d3.wait()

        def acc_step(l, accs):
            a0, a1 = accs
            s = idfs_v[l, 0]
            a0 = a0 + rows_v[l, pl.ds(0, 16)] * s
            a1 = a1 + rows_v[l, pl.ds(16, 16)] * s
            return (a0, a1)

        zero = jnp.zeros((16,), jnp.float32)
        a0, a1 = lax.fori_loop(0, L, acc_step, (zero, zero), unroll=4)
        inv = 1.0 / len_v[r].astype(jnp.float32)
        out_v[r, pl.ds(0, 16)] = a0 * inv
        out_v[r, pl.ds(16, 16)] = a1 * inv
        return carry

    lax.fori_loop(0, RPW, do_row, 0)
    pltpu.sync_copy(out_v, out_hbm.at[pl.ds(base, RPW)])


@jax.jit
def _run(text_flat, text_len, idf, table):
    mesh = plsc.VectorSubcoreMesh(core_axis_name="c", subcore_axis_name="s")
    f = pl.kernel(
        _body,
        out_type=jax.ShapeDtypeStruct((B, D), jnp.float32),
        mesh=mesh,
        scratch_types=[
            pltpu.VMEM((RPW * L,), jnp.int32),
            pltpu.VMEM((RPW,), jnp.int32),
            pltpu.VMEM((L, D), jnp.float32),
            pltpu.VMEM((L, 1), jnp.float32),
            pltpu.VMEM((RPW, D), jnp.float32),
            pltpu.SemaphoreType.DMA,
        ],
    )
    return f(text_flat, text_len, idf, table)


def kernel(text, text_len, idf, table):
    return _run(text.reshape(-1), text_len, idf, table)


# speedup vs baseline: 2.1187x; 2.1187x over previous
import jax
import jax.numpy as jnp
from jax import lax
from jax.experimental import pallas as pl
from jax.experimental.pallas import tpu as pltpu
from jax.experimental.pallas import tpu_sc as plsc

B = 4096
L = 200
LP = 208
D = 32
NW = 32
RPW = B // NW
C0 = 104
C1 = L - C0


def _body(text_hbm, len_hbm, idf_hbm, table_hbm, out_hbm,
          text_v, len_v, rows_v, idfs_v, out_v, sem):
    wid = lax.axis_index("s") * 2 + lax.axis_index("c")
    base = wid * RPW

    pltpu.sync_copy(text_hbm.at[pl.ds(base * L, RPW * L)], text_v)
    pltpu.sync_copy(len_hbm.at[pl.ds(base, RPW)], len_v)

    z = jnp.zeros((16,), jnp.float32)
    idfs_v[pl.ds(LP - 16, 16)] = z
    for lpad in range(L, LP):
        rows_v[lpad, pl.ds(0, 16)] = z
        rows_v[lpad, pl.ds(16, 16)] = z

    def do_block(rb, carry):
        inv = 1.0 / len_v[pl.ds(rb * 16, 16)].astype(jnp.float32)
        for j in range(16):
            r = rb * 16 + j
            toff = r * L
            i0 = text_v.at[pl.ds(toff, C0)]
            i1 = text_v.at[pl.ds(toff + C0, C1)]
            d0 = pltpu.async_copy(table_hbm.at[i0], rows_v.at[pl.ds(0, C0)], sem)
            d1 = pltpu.async_copy(table_hbm.at[i1], rows_v.at[pl.ds(C0, C1)], sem)
            d2 = pltpu.async_copy(idf_hbm.at[i0], idfs_v.at[pl.ds(0, C0)], sem)
            d3 = pltpu.async_copy(idf_hbm.at[i1], idfs_v.at[pl.ds(C0, C1)], sem)
            d0.wait()
            d1.wait()
            d2.wait()
            d3.wait()

            def acc_step(lb, accs):
                a0, a1 = accs
                w = idfs_v[pl.ds(lb * 16, 16)]
                for jj in range(16):
                    l = lb * 16 + jj
                    s = w[jj]
                    a0 = a0 + rows_v[l, pl.ds(0, 16)] * s
                    a1 = a1 + rows_v[l, pl.ds(16, 16)] * s
                return (a0, a1)

            a0, a1 = lax.fori_loop(0, LP // 16, acc_step, (z, z))
            siv = inv[j]
            out_v[r, pl.ds(0, 16)] = a0 * siv
            out_v[r, pl.ds(16, 16)] = a1 * siv
        return carry

    lax.fori_loop(0, RPW // 16, do_block, 0)
    pltpu.sync_copy(out_v, out_hbm.at[pl.ds(base, RPW)])


@jax.jit
def _run(text_flat, text_len, idf_flat, table):
    mesh = plsc.VectorSubcoreMesh(core_axis_name="c", subcore_axis_name="s")
    f = pl.kernel(
        _body,
        out_type=jax.ShapeDtypeStruct((B, D), jnp.float32),
        mesh=mesh,
        compiler_params=pltpu.CompilerParams(use_tc_tiling_on_sc=False),
        scratch_types=[
            pltpu.VMEM((RPW * L,), jnp.int32),
            pltpu.VMEM((RPW,), jnp.int32),
            pltpu.VMEM((LP, D), jnp.float32),
            pltpu.VMEM((LP,), jnp.float32),
            pltpu.VMEM((RPW, D), jnp.float32),
            pltpu.SemaphoreType.DMA,
        ],
    )
    return f(text_flat, text_len, idf_flat, table)


def kernel(text, text_len, idf, table):
    return _run(text.reshape(-1), text_len, idf.reshape(-1), table)


# speedup vs baseline: 2.4057x; 1.1354x over previous
import jax
import jax.numpy as jnp
from jax import lax
from jax.experimental import pallas as pl
from jax.experimental.pallas import tpu as pltpu
from jax.experimental.pallas import tpu_sc as plsc

B = 4096
L = 200
LP = 208
D = 32
NW = 32
RPW = B // NW
C0 = 104
C1 = L - C0


def _body(text_hbm, len_hbm, idf_hbm, table_hbm, out_hbm,
          text_v, len_v, rows0, rows1, idfs0, idfs1, out_v, sem0, sem1):
    wid = lax.axis_index("s") * 2 + lax.axis_index("c")
    base = wid * RPW
    bufs = ((rows0, idfs0, sem0), (rows1, idfs1, sem1))

    pltpu.sync_copy(text_hbm.at[pl.ds(base * L, RPW * L)], text_v)
    pltpu.sync_copy(len_hbm.at[pl.ds(base, RPW)], len_v)

    z = jnp.zeros((16,), jnp.float32)
    for rv, iv, _ in bufs:
        iv[pl.ds(LP - 16, 16)] = z
        for lpad in range(L, LP):
            rv[lpad, pl.ds(0, 16)] = z
            rv[lpad, pl.ds(16, 16)] = z

    def fire(r, slot):
        rv, iv, sem = bufs[slot]
        i0 = text_v.at[pl.ds(r * L, C0)]
        i1 = text_v.at[pl.ds(r * L + C0, C1)]
        pltpu.async_copy(table_hbm.at[i0], rv.at[pl.ds(0, C0)], sem)
        pltpu.async_copy(table_hbm.at[i1], rv.at[pl.ds(C0, C1)], sem)
        pltpu.async_copy(idf_hbm.at[i0], iv.at[pl.ds(0, C0)], sem)
        pltpu.async_copy(idf_hbm.at[i1], iv.at[pl.ds(C0, C1)], sem)

    def wait_slot(slot):
        rv, iv, sem = bufs[slot]
        pltpu.make_async_copy(table_hbm.at[pl.ds(0, C0)], rv.at[pl.ds(0, C0)], sem).wait()
        pltpu.make_async_copy(table_hbm.at[pl.ds(0, C1)], rv.at[pl.ds(C0, C1)], sem).wait()
        pltpu.make_async_copy(idf_hbm.at[pl.ds(0, C0)], iv.at[pl.ds(0, C0)], sem).wait()
        pltpu.make_async_copy(idf_hbm.at[pl.ds(0, C1)], iv.at[pl.ds(C0, C1)], sem).wait()

    fire(0, 0)

    def do_block(rb, carry):
        inv = 1.0 / len_v[pl.ds(rb * 16, 16)].astype(jnp.float32)
        for j in range(16):
            r = rb * 16 + j
            slot = j % 2
            rv, iv, _ = bufs[slot]

            @pl.when(r + 1 < RPW)
            def _():
                fire(r + 1, (j + 1) % 2)

            wait_slot(slot)

            def acc_step(lb, accs):
                a0, a1 = accs
                w = iv[pl.ds(lb * 16, 16)]
                for jj in range(16):
                    l = lb * 16 + jj
                    s = w[jj]
                    a0 = a0 + rv[l, pl.ds(0, 16)] * s
                    a1 = a1 + rv[l, pl.ds(16, 16)] * s
                return (a0, a1)

            a0, a1 = lax.fori_loop(0, LP // 16, acc_step, (z, z))
            siv = inv[j]
            out_v[r, pl.ds(0, 16)] = a0 * siv
            out_v[r, pl.ds(16, 16)] = a1 * siv
        return carry

    lax.fori_loop(0, RPW // 16, do_block, 0)
    pltpu.sync_copy(out_v, out_hbm.at[pl.ds(base, RPW)])


@jax.jit
def _run(text_flat, text_len, idf_flat, table):
    mesh = plsc.VectorSubcoreMesh(core_axis_name="c", subcore_axis_name="s")
    f = pl.kernel(
        _body,
        out_type=jax.ShapeDtypeStruct((B, D), jnp.float32),
        mesh=mesh,
        compiler_params=pltpu.CompilerParams(use_tc_tiling_on_sc=False),
        scratch_types=[
            pltpu.VMEM((RPW * L,), jnp.int32),
            pltpu.VMEM((RPW,), jnp.int32),
            pltpu.VMEM((LP, D), jnp.float32),
            pltpu.VMEM((LP, D), jnp.float32),
            pltpu.VMEM((LP,), jnp.float32),
            pltpu.VMEM((LP,), jnp.float32),
            pltpu.VMEM((RPW, D), jnp.float32),
            pltpu.SemaphoreType.DMA,
            pltpu.SemaphoreType.DMA,
        ],
    )
    return f(text_flat, text_len, idf_flat, table)


def kernel(text, text_len, idf, table):
    return _run(text.reshape(-1), text_len, idf.reshape(-1), table)
